# Initial kernel scaffold; baseline (speedup 1.0000x reference)
import jax
import jax.numpy as jnp
from jax.experimental import pallas as pl


def kernel(x, y):
    raise NotImplementedError("write your pallas kernel here")


# speedup vs baseline: 150.4690x; 150.4690x over previous
import jax
import jax.numpy as jnp
from jax import lax
from jax.experimental import pallas as pl
from jax.experimental.pallas import tpu as pltpu
from jax.experimental.pallas import tpu_sc as plsc

N = 2048
C = 128
L = 16
NCH = N // L
NC, NS = 2, 16
NW = NC * NS
CPW = C // NW
EPS = 1e-10


def _wd_body(x_hbm, y_hbm, out_hbm, f_v, g_v, gp_v, gn_v, res_v):
    cid = lax.axis_index("c")
    sid = lax.axis_index("s")
    wid = sid * NC + cid
    iota_f = lax.iota(jnp.int32, L).astype(jnp.float32)
    z = jnp.float32(0)

    acc = jnp.zeros((L,), jnp.float32)
    for j in range(CPW):
        col = wid * CPW + j
        pltpu.sync_copy(x_hbm.at[col], f_v)
        pltpu.sync_copy(y_hbm.at[col], g_v)

        def sums_body(i, carry):
            wp, wn, gp, gn = carry
            fc = f_v[pl.ds(i * L, L)]
            gc = g_v[pl.ds(i * L, L)]
            wp = wp + jnp.sum(jnp.maximum(fc, 0.0))
            wn = wn + jnp.sum(jnp.maximum(-fc, 0.0))
            gp = gp + jnp.sum(jnp.maximum(gc, 0.0))
            gn = gn + jnp.sum(jnp.maximum(-gc, 0.0))
            return wp, wn, gp, gn

        wp_s, wn_s, gp_s, gn_s = lax.fori_loop(
            0, NCH, sums_body, (z, z, z, z))

        def build_body(i, carry):
            cgp, cgn = carry
            gc = g_v[pl.ds(i * L, L)]
            gpc = jnp.maximum(gc, 0.0) / gp_s
            gnc = jnp.maximum(-gc, 0.0) / gn_s
            gp_v[pl.ds(i * L, L)] = plsc.cumsum(gpc) + cgp
            gn_v[pl.ds(i * L, L)] = plsc.cumsum(gnc) + cgn
            return cgp + jnp.sum(gpc), cgn + jnp.sum(gnc)

        lax.fori_loop(0, NCH, build_body, (z, z))

        for sgn, G_ref, w_s in ((1.0, gp_v, wp_s), (-1.0, gn_v, wn_s)):
            def query_body(i, carry, sgn=sgn, G_ref=G_ref, w_s=w_s):
                cf, a = carry
                fc = f_v[pl.ds(i * L, L)]
                wc = jnp.maximum(fc * sgn, 0.0) / w_s
                q = plsc.cumsum(wc) + cf
                cf = cf + jnp.sum(wc)
                t = (i * L).astype(jnp.float32) + iota_f
                res = jnp.zeros((L,), jnp.int32)
                step = N // 2
                while step >= 1:
                    probe = plsc.load_gather(G_ref, [res + (step - 1)])
                    res = jnp.where(probe < q, res + step, res)
                    step //= 2
                fin = plsc.load_gather(G_ref, [res])
                res = jnp.where(fin < q, res + 1, res)
                idx = jnp.clip(res - 1, 0, N - 2)
                g0 = plsc.load_gather(G_ref, [idx])
                g1 = plsc.load_gather(G_ref, [idx + 1])
                frac = (q - g0) / (g1 - g0 + EPS)
                diff = t - (idx.astype(jnp.float32) + frac)
                return cf, a + diff * diff * wc

            _, acc = lax.fori_loop(0, NCH, query_body, (z, acc))

    res_v[...] = acc
    pltpu.sync_copy(res_v, out_hbm.at[wid])


_sc_call = pl.kernel(
    _wd_body,
    out_type=jax.ShapeDtypeStruct((NW, L), jnp.float32),
    mesh=plsc.VectorSubcoreMesh(core_axis_name="c", subcore_axis_name="s"),
    compiler_params=pltpu.CompilerParams(needs_layout_passes=False),
    scratch_types=[
        pltpu.VMEM((N,), jnp.float32),
        pltpu.VMEM((N,), jnp.float32),
        pltpu.VMEM((N,), jnp.float32),
        pltpu.VMEM((N,), jnp.float32),
        pltpu.VMEM((L,), jnp.float32),
    ],
)


def kernel(x, y):
    xT = x.reshape(N, C).T
    yT = y.reshape(N, C).T
    part = _sc_call(xT, yT)
    return jnp.sum(part)


# speedup vs baseline: 170.9421x; 1.1361x over previous
import jax
import jax.numpy as jnp
from jax import lax
from jax.experimental import pallas as pl
from jax.experimental.pallas import tpu as pltpu
from jax.experimental.pallas import tpu_sc as plsc

N = 2048
C = 128
L = 16
NCH = N // L
NC, NS = 2, 16
NW = NC * NS
CPW = C // NW
UNROLL = 2
EPS = 1e-10


def _search_interp(G_ref, q, wc, t):
    res = jnp.zeros((L,), jnp.int32)
    step = N // 2
    while step >= 1:
        probe = plsc.load_gather(G_ref, [res + (step - 1)])
        res = jnp.where(probe < q, res + step, res)
        step //= 2
    idx = jnp.clip(res - 1, 0, N - 2)
    g0 = plsc.load_gather(G_ref, [idx])
    g1 = plsc.load_gather(G_ref, [idx + 1])
    frac = (q - g0) / (g1 - g0 + EPS)
    diff = t - (idx.astype(jnp.float32) + frac)
    return diff * diff * wc


def _wd_body(x_hbm, y_hbm, out_hbm, f_v, g_v, gp_v, gn_v, res_v):
    cid = lax.axis_index("c")
    sid = lax.axis_index("s")
    wid = sid * NC + cid
    iota_f = lax.iota(jnp.int32, L).astype(jnp.float32)
    z = jnp.float32(0)
    zv = jnp.zeros((L,), jnp.float32)

    acc = jnp.zeros((L,), jnp.float32)
    for j in range(CPW):
        col = wid * CPW + j
        pltpu.sync_copy(x_hbm.at[col], f_v)
        pltpu.sync_copy(y_hbm.at[col], g_v)

        def sums_body(i, carry):
            wp, wn, gp, gn = carry
            fc = f_v[pl.ds(i * L, L)]
            gc = g_v[pl.ds(i * L, L)]
            return (wp + jnp.maximum(fc, 0.0), wn + jnp.maximum(-fc, 0.0),
                    gp + jnp.maximum(gc, 0.0), gn + jnp.maximum(-gc, 0.0))

        wp_v, wn_v, gp_acc, gn_acc = lax.fori_loop(
            0, NCH, sums_body, (zv, zv, zv, zv))
        one_v = 1.0 + zv
        rwp = one_v / (jnp.sum(wp_v) + zv)
        rwn = one_v / (jnp.sum(wn_v) + zv)
        rgp = one_v / (jnp.sum(gp_acc) + zv)
        rgn = one_v / (jnp.sum(gn_acc) + zv)

        def build_body(i, carry):
            cgp, cgn = carry
            gc = g_v[pl.ds(i * L, L)]
            gpc = jnp.maximum(gc, 0.0) * rgp
            gnc = jnp.maximum(-gc, 0.0) * rgn
            gp_v[pl.ds(i * L, L)] = plsc.cumsum(gpc) + cgp
            gn_v[pl.ds(i * L, L)] = plsc.cumsum(gnc) + cgn
            return cgp + jnp.sum(gpc), cgn + jnp.sum(gnc)

        lax.fori_loop(0, NCH, build_body, (z, z))

        def chunk_terms(k, cfp, cfn):
            fc = f_v[pl.ds(k * L, L)]
            wcp = jnp.maximum(fc, 0.0) * rwp
            wcn = jnp.maximum(-fc, 0.0) * rwn
            qp = plsc.cumsum(wcp) + cfp
            qn = plsc.cumsum(wcn) + cfn
            t = (k * L).astype(jnp.float32) + iota_f
            terms = (_search_interp(gp_v, qp, wcp, t)
                     + _search_interp(gn_v, qn, wcn, t))
            return terms, cfp + jnp.sum(wcp), cfn + jnp.sum(wcn)

        def query_body(i, carry):
            cfp, cfn, a = carry
            for u in range(UNROLL):
                terms, cfp, cfn = chunk_terms(i * UNROLL + u, cfp, cfn)
                a = a + terms
            return cfp, cfn, a

        _, _, acc = lax.fori_loop(0, NCH // UNROLL, query_body, (z, z, acc))

    res_v[...] = acc
    pltpu.sync_copy(res_v, out_hbm.at[wid])


_sc_call = pl.kernel(
    _wd_body,
    out_type=jax.ShapeDtypeStruct((NW, L), jnp.float32),
    mesh=plsc.VectorSubcoreMesh(core_axis_name="c", subcore_axis_name="s"),
    compiler_params=pltpu.CompilerParams(needs_layout_passes=False),
    scratch_types=[
        pltpu.VMEM((N,), jnp.float32),
        pltpu.VMEM((N,), jnp.float32),
        pltpu.VMEM((N,), jnp.float32),
        pltpu.VMEM((N,), jnp.float32),
        pltpu.VMEM((L,), jnp.float32),
    ],
)


def kernel(x, y):
    xT = x.reshape(N, C).T
    yT = y.reshape(N, C).T
    part = _sc_call(xT, yT)
    return jnp.sum(part)
